# baseline (speedup 1.0000x reference)
import functools

import jax
import jax.numpy as jnp
from jax import lax
from jax.experimental import pallas as pl
from jax.experimental.pallas import tpu as pltpu
from jax.experimental.pallas import tpu_sc as plsc


def _embed_lookup(n, d, n_chunks, chunk, nbuf):
    mesh = plsc.VectorSubcoreMesh(core_axis_name="c", subcore_axis_name="s")
    assert n_chunks % nbuf == 0
    n_groups = n_chunks // nbuf

    scratch = [pltpu.VMEM((n_chunks, chunk), jnp.int32)]
    scratch += [pltpu.VMEM((chunk, d), jnp.float32) for _ in range(nbuf)]
    scratch += [pltpu.SemaphoreType.DMA for _ in range(2 * nbuf)]

    @functools.partial(
        pl.kernel,
        mesh=mesh,
        out_type=jax.ShapeDtypeStruct((n, 128), jnp.float32),
        compiler_params=pltpu.CompilerParams(use_tc_tiling_on_sc=False),
        scratch_types=scratch,
    )
    def k(table_h, idx_h, out_h, idx_v, *rest):
        bufs = rest[:nbuf]
        gsems = rest[nbuf : 2 * nbuf]
        ssems = rest[2 * nbuf :]
        nc = plsc.get_sparse_core_info().num_cores
        wid = lax.axis_index("s") * nc + lax.axis_index("c")
        pltpu.sync_copy(idx_h.at[wid], idx_v)
        base = wid * (n_chunks * chunk)

        def gather(g, b):
            return pltpu.make_async_copy(table_h.at[idx_v.at[g]], bufs[b], gsems[b])

        def store(g, b):
            return pltpu.make_async_copy(
                bufs[b],
                out_h.at[pl.ds(base + g * chunk, chunk), pl.ds(0, d)],
                ssems[b],
            )

        for b in range(nbuf):
            gather(b, b).start()

        def group(i, carry):
            g0 = i * nbuf
            for b in range(nbuf):
                gather(g0 + b, b).wait()
                store(g0 + b, b).start()
            for b in range(nbuf):
                store(g0 + b, b).wait()
                ng = g0 + nbuf + b

                @pl.when(ng < n_chunks)
                def _():
                    gather(ng, b).start()

            return carry

        lax.fori_loop(0, n_groups, group, 0)

    return k


def kernel(text, table):
    b, h = text.shape
    v, d = table.shape
    n = b * h
    info = plsc.get_sparse_core_info()
    nw = info.num_cores * info.num_subcores
    chunk = 128
    assert n % (nw * chunk) == 0
    n_chunks = n // (nw * chunk)
    table2 = jnp.pad(table, ((0, 0), (0, 128 - d))).reshape(2 * v, d)
    idx = (text * 2).reshape(nw, n_chunks, chunk)
    out = _embed_lookup(n, d, n_chunks, chunk, nbuf=10)(table2, idx)
    return out.reshape(b, h, 128)[:, :, :d]


# speedup vs baseline: 1.0005x; 1.0005x over previous
import functools

import jax
import jax.numpy as jnp
from jax import lax
from jax.experimental import pallas as pl
from jax.experimental.pallas import tpu as pltpu
from jax.experimental.pallas import tpu_sc as plsc


def _embed_lookup(n, d, n_chunks, chunk, nbuf):
    mesh = plsc.VectorSubcoreMesh(core_axis_name="c", subcore_axis_name="s")
    assert n_chunks % nbuf == 0
    n_groups = n_chunks // nbuf

    assert nbuf % 2 == 0
    scratch = [pltpu.VMEM((n_chunks, chunk), jnp.int32)]
    scratch += [pltpu.VMEM((2 * chunk, d), jnp.float32) for _ in range(nbuf // 2)]
    scratch += [pltpu.SemaphoreType.DMA for _ in range(nbuf + nbuf // 2)]

    @functools.partial(
        pl.kernel,
        mesh=mesh,
        out_type=jax.ShapeDtypeStruct((n, 128), jnp.float32),
        compiler_params=pltpu.CompilerParams(use_tc_tiling_on_sc=False),
        scratch_types=scratch,
    )
    def k(table_h, idx_h, out_h, idx_v, *rest):
        bufs = rest[: nbuf // 2]
        gsems = rest[nbuf // 2 : nbuf // 2 + nbuf]
        ssems = rest[nbuf // 2 + nbuf :]
        nc = plsc.get_sparse_core_info().num_cores
        wid = lax.axis_index("s") * nc + lax.axis_index("c")
        pltpu.sync_copy(idx_h.at[wid], idx_v)
        base = wid * (n_chunks * chunk)

        def gather(g, b):
            dst = bufs[b // 2].at[pl.ds((b % 2) * chunk, chunk), :]
            return pltpu.make_async_copy(table_h.at[idx_v.at[g]], dst, gsems[b])

        def store(g0, p):
            return pltpu.make_async_copy(
                bufs[p],
                out_h.at[pl.ds(base + g0 * chunk, 2 * chunk), pl.ds(0, d)],
                ssems[p],
            )

        for b in range(nbuf):
            gather(b, b).start()

        def group(i, carry):
            g0 = i * nbuf
            for b in range(0, nbuf, 2):
                gather(g0 + b, b).wait()
                gather(g0 + b + 1, b + 1).wait()
                store(g0 + b, b // 2).start()
            for b in range(0, nbuf, 2):
                store(g0 + b, b // 2).wait()
                for bb in (b, b + 1):
                    ng = g0 + nbuf + bb

                    @pl.when(ng < n_chunks)
                    def _():
                        gather(ng, bb).start()

            return carry

        lax.fori_loop(0, n_groups, group, 0)

    return k


def kernel(text, table):
    b, h = text.shape
    v, d = table.shape
    n = b * h
    info = plsc.get_sparse_core_info()
    nw = info.num_cores * info.num_subcores
    chunk = 128
    assert n % (nw * chunk) == 0
    n_chunks = n // (nw * chunk)
    table2 = jnp.pad(table, ((0, 0), (0, 128 - d))).reshape(2 * v, d)
    idx = (text * 2).reshape(nw, n_chunks, chunk)
    out = _embed_lookup(n, d, n_chunks, chunk, nbuf=10)(table2, idx)
    return out.reshape(b, h, 128)[:, :, :d]
